# baseline (speedup 1.0000x reference)
import functools

import jax
import jax.numpy as jnp
from jax import lax
from jax.experimental import pallas as pl
from jax.experimental.pallas import tpu as pltpu
from jax.experimental.pallas import tpu_sc as plsc

D = 10000
B = 8
V2 = 2500
K = 25
LMAX = 2.0

DP = 2560
PASS_ROWS = 640
PASS_WORDS = PASS_ROWS * DP
TILE_WORDS = PASS_WORDS // 16
NE2 = V2 * 16 + V2
NE2_T = 2688
NC2 = NE2_T // 128


def _combine1_body(a_ref, w_ref, b_ref, o_ref):
    res = jax.lax.dot_general(a_ref[...], w_ref[...], (((1,), (1,)), ((), ())),
                              preferred_element_type=jnp.float32)
    res = jnp.maximum(res + b_ref[...], 0.0)
    y = res.reshape(250, 32, 32)
    m = jnp.maximum(jnp.maximum(y[:, 0:8], y[:, 8:16]),
                    jnp.maximum(y[:, 16:24], y[:, 24:32]))
    o_ref[...] = m.reshape(2000, 32)


def _tc_combine1(a, w, b):
    return pl.pallas_call(
        _combine1_body,
        grid=(10,),
        in_specs=[
            pl.BlockSpec((8000, 25), lambda i: (i, 0)),
            pl.BlockSpec((32, 25), lambda i: (0, 0)),
            pl.BlockSpec((1, 32), lambda i: (0, 0)),
        ],
        out_specs=pl.BlockSpec((2000, 32), lambda i: (i, 0)),
        out_shape=jax.ShapeDtypeStruct((20000, 32), jnp.float32),
    )(a, w, b)


def _combine2_body(a_ref, w_ref, b_ref, o_ref):
    res = jax.lax.dot_general(a_ref[...], w_ref[...], (((1,), (1,)), ((), ())),
                              preferred_element_type=jnp.float32)
    res = jnp.maximum(res + b_ref[...], 0.0)
    y = res.reshape(1000, 4, 64)
    o_ref[...] = jnp.max(y, axis=1)


def _tc_combine2(a, w, b):
    return pl.pallas_call(
        _combine2_body,
        grid=(5,),
        in_specs=[
            pl.BlockSpec((4000, 800), lambda i: (i, 0)),
            pl.BlockSpec((64, 800), lambda i: (0, 0)),
            pl.BlockSpec((1, 64), lambda i: (0, 0)),
        ],
        out_specs=pl.BlockSpec((1000, 64), lambda i: (i, 0)),
        out_shape=jax.ShapeDtypeStruct((5000, 64), jnp.float32),
    )(a, w, b)


def _fc_body(h_ref, w1_ref, b1_ref, w2_ref, b2_ref, o_ref, zt_ref):
    i = pl.program_id(0)
    zt = jnp.maximum(jax.lax.dot_general(
        w1_ref[...], h_ref[...], (((1,), (1,)), ((), ())),
        preferred_element_type=jnp.float32) + b1_ref[...], 0.0)
    zt_ref[pl.ds(i * 64, 64), :] = zt

    @pl.when(i == pl.num_programs(0) - 1)
    def _():
        o_ref[...] = jax.lax.dot_general(
            zt_ref[...], w2_ref[...], (((0,), (1,)), ((), ())),
            preferred_element_type=jnp.float32) + b2_ref[...]


def _tc_fc(h, w1, b1, w2, b2):
    from jax.experimental.pallas import tpu as pltpu
    return pl.pallas_call(
        _fc_body,
        grid=(8,),
        in_specs=[
            pl.BlockSpec((8, 40000), lambda i: (0, 0)),
            pl.BlockSpec((64, 40000), lambda i: (i, 0)),
            pl.BlockSpec((64, 1), lambda i: (i, 0)),
            pl.BlockSpec((10, 512), lambda i: (0, 0)),
            pl.BlockSpec((1, 10), lambda i: (0, 0)),
        ],
        out_specs=pl.BlockSpec((8, 10), lambda i: (0, 0)),
        out_shape=jax.ShapeDtypeStruct((8, 10), jnp.float32),
        scratch_shapes=[pltpu.VMEM((512, 8), jnp.float32)],
    )(h, w1, b1, w2, b2)


def _densify_sc(li_all, vals_tiled, zeros_flat):
    mesh = plsc.VectorSubcoreMesh(core_axis_name="c", subcore_axis_name="s")

    @functools.partial(
        pl.kernel, mesh=mesh,
        out_type=jax.ShapeDtypeStruct((DP * DP,), jnp.float32),
        scratch_types=[
            pltpu.VMEM((NC2, 128), jnp.int32),
            pltpu.VMEM((NE2_T,), jnp.float32),
            pltpu.VMEM_SHARED((PASS_WORDS,), jnp.float32),
        ],
    )
    def run(li_hbm, vals_hbm, zeros_hbm, m_hbm, libuf, vbuf, spbuf):
        c = lax.axis_index("c")
        s = lax.axis_index("s")
        pltpu.sync_copy(vals_hbm.at[s], vbuf)
        tile_off = s * TILE_WORDS
        for p_local in range(2):
            pglob = 2 * c + p_local
            pltpu.sync_copy(zeros_hbm.at[pl.ds(tile_off, TILE_WORDS)],
                            spbuf.at[pl.ds(tile_off, TILE_WORDS)])
            pltpu.sync_copy(li_hbm.at[pglob, s], libuf)
            plsc.subcore_barrier()

            def scat(ci, carry):
                pltpu.sync_copy(vbuf.at[pl.ds(ci * 128, 128)],
                                spbuf.at[libuf.at[ci]], add=True)
                return carry

            lax.fori_loop(0, NC2, scat, 0)
            plsc.subcore_barrier()
            pltpu.sync_copy(
                spbuf.at[pl.ds(tile_off, TILE_WORDS)],
                m_hbm.at[pl.ds(pglob * PASS_WORDS + tile_off, TILE_WORDS)])
            plsc.subcore_barrier()

    return run(li_all, vals_tiled, zeros_flat)


def _densify_cl2(l2_idx, l2_vals):
    rows = jnp.concatenate([l2_idx[0], jnp.arange(V2, dtype=jnp.int32)])
    cols = jnp.concatenate([l2_idx[1], jnp.arange(V2, dtype=jnp.int32)])
    vals = jnp.concatenate([l2_vals, jnp.full((V2,), -1.0, jnp.float32)])
    npad = 16 * NE2_T - NE2
    rows = jnp.concatenate([rows, jnp.zeros((npad,), jnp.int32)])
    cols = jnp.concatenate([cols, jnp.zeros((npad,), jnp.int32)])
    vals = jnp.concatenate([vals, jnp.zeros((npad,), jnp.float32)])
    tile_id = (jnp.arange(16 * NE2_T, dtype=jnp.int32) // NE2_T) % 16
    base = (jnp.arange(4, dtype=jnp.int32) * PASS_ROWS)[:, None]
    in_pass = (rows[None, :] >= base) & (rows[None, :] < base + PASS_ROWS)
    li = jnp.where(in_pass, (rows[None, :] - base) * DP + cols[None, :],
                   2512 + tile_id[None, :])
    li_all = li.reshape(4, 16, NC2, 128)
    vals_tiled = vals.reshape(16, NE2_T)
    zeros_flat = jnp.zeros((PASS_WORDS,), jnp.float32)
    m_flat = _densify_sc(li_all, vals_tiled, zeros_flat)
    return m_flat.reshape(DP, DP)


_CH = 512


def _cheby_dense_body(lh_ref, ll_ref, x0_ref, wk_ref, b2_ref, o_ref,
                      xa_ref, xb_ref, xh_ref, xl_ref, acc_ref):
    p = pl.program_id(0)
    dims = (((1,), (0,)), ((), ()))

    def split(src_ref):
        for r in range(DP // _CH):
            sl = pl.ds(r * _CH, _CH)
            chunk = src_ref[sl, :]
            h = chunk.astype(jnp.bfloat16)
            xh_ref[sl, :] = h
            xl_ref[sl, :] = (chunk - h.astype(jnp.float32)).astype(jnp.bfloat16)

    def mm_rows(r):
        sl = pl.ds(r * _CH, _CH)
        acc = jax.lax.dot_general(lh_ref[sl, :], xh_ref[...], dims,
                                  preferred_element_type=jnp.float32)
        acc += jax.lax.dot_general(lh_ref[sl, :], xl_ref[...], dims,
                                   preferred_element_type=jnp.float32)
        acc += jax.lax.dot_general(ll_ref[sl, :], xh_ref[...], dims,
                                   preferred_element_type=jnp.float32)
        return acc

    def add_contrib(x, kk):
        w = wk_ref[kk]
        for b in range(B):
            acc_ref[b] += jax.lax.dot_general(
                x[:, b * 32:(b + 1) * 32], w, dims,
                preferred_element_type=jnp.float32)

    @pl.when(p == 0)
    def _():
        acc_ref[...] = jnp.zeros_like(acc_ref)
        x0 = x0_ref[...]
        xa_ref[...] = x0
        add_contrib(x0, 0)
        split(x0_ref)
        for r in range(DP // _CH):
            sl = pl.ds(r * _CH, _CH)
            xn = mm_rows(r)
            xb_ref[sl, :] = xn
        add_contrib(xb_ref[...], 1)

    @pl.when(p % 2 == 1)
    def _():
        split(xb_ref)
        for r in range(DP // _CH):
            sl = pl.ds(r * _CH, _CH)
            xn = 2.0 * mm_rows(r) - xa_ref[sl, :]
            xa_ref[sl, :] = xn
        add_contrib(xa_ref[...], p + 1)

    @pl.when((p % 2 == 0) & (p > 0))
    def _():
        split(xa_ref)
        for r in range(DP // _CH):
            sl = pl.ds(r * _CH, _CH)
            xn = 2.0 * mm_rows(r) - xb_ref[sl, :]
            xb_ref[sl, :] = xn
        add_contrib(xb_ref[...], p + 1)

    @pl.when(p == K - 2)
    def _():
        for b in range(B):
            z = jnp.maximum(acc_ref[b][:V2, :] + b2_ref[...], 0.0)
            o_ref[b] = jnp.max(z.reshape(V2 // 4, 4, 64), axis=1)


def _tc_cheby_dense(m, x0, wk, b2):
    mh = m.astype(jnp.bfloat16)
    ml = (m - mh.astype(jnp.float32)).astype(jnp.bfloat16)
    return pl.pallas_call(
        _cheby_dense_body,
        grid=(K - 1,),
        in_specs=[
            pl.BlockSpec((DP, DP), lambda p: (0, 0)),
            pl.BlockSpec((DP, DP), lambda p: (0, 0)),
            pl.BlockSpec((DP, 256), lambda p: (0, 0)),
            pl.BlockSpec((K, 32, 64), lambda p: (0, 0, 0)),
            pl.BlockSpec((1, 64), lambda p: (0, 0)),
        ],
        out_specs=pl.BlockSpec((B, V2 // 4, 64), lambda p: (0, 0, 0)),
        out_shape=jax.ShapeDtypeStruct((B, V2 // 4, 64), jnp.float32),
        scratch_shapes=[pltpu.VMEM((DP, 256), jnp.float32),
                        pltpu.VMEM((DP, 256), jnp.float32),
                        pltpu.VMEM((DP, 256), jnp.bfloat16),
                        pltpu.VMEM((DP, 256), jnp.bfloat16),
                        pltpu.VMEM((B, DP, 64), jnp.float32)],
    )(mh, ml, x0, wk, b2)


V1P = 10112
RT1 = V1P // 16
SLW = RT1 * 4
TW1 = V1P * 4
NE1_T = 10112
EW1 = NE1_T * 4
NSC1 = EW1 // 128
_SUP = 8192


def _cl1_sc(x0sc, cols2, vals2, sidx3):
    mesh = plsc.VectorSubcoreMesh(core_axis_name="c", subcore_axis_name="s")

    @functools.partial(
        pl.kernel, mesh=mesh,
        out_type=jax.ShapeDtypeStruct((2 * K * TW1,), jnp.float32),
        compiler_params=pltpu.CompilerParams(needs_layout_passes=False),
        scratch_types=[
            pltpu.VMEM((NE1_T,), jnp.int32),
            pltpu.VMEM((NE1_T,), jnp.float32),
            pltpu.VMEM((NSC1, 128), jnp.int32),
            pltpu.VMEM((TW1,), jnp.float32),
            pltpu.VMEM((_SUP,), jnp.float32),
            pltpu.VMEM((SLW,), jnp.float32),
            pltpu.VMEM((SLW,), jnp.float32),
            pltpu.VMEM((SLW,), jnp.float32),
            pltpu.VMEM((SLW,), jnp.float32),
            pltpu.VMEM_SHARED((TW1,), jnp.float32),
            pltpu.VMEM_SHARED((TW1,), jnp.float32),
            pltpu.SemaphoreType.DMA,
        ],
    )
    def run(x_hbm, cols_hbm, vals_hbm, sidx_hbm, out_hbm,
            cols_t, vals_t, sidx_t, z_t, gbuf, ybuf, xm1_t, xm2_t, zzero,
            z_s, y_s, dsem):
        c = lax.axis_index("c")
        s = lax.axis_index("s")
        sl4 = pl.ds(s * SLW, SLW)

        iota16 = lax.iota(jnp.int32, 16)
        br = iota16 // 4
        bc4 = iota16 % 4
        zero16 = jnp.zeros((16,), jnp.float32)

        pltpu.sync_copy(cols_hbm.at[pl.ds(s * NE1_T, NE1_T)], cols_t)
        pltpu.sync_copy(vals_hbm.at[pl.ds(s * NE1_T, NE1_T)], vals_t)
        pltpu.sync_copy(sidx_hbm.at[s], sidx_t)

        def zinit(j, carry):
            zzero[pl.ds(j * 16, 16)] = zero16
            return carry
        lax.fori_loop(0, SLW // 16, zinit, 0)

        pltpu.sync_copy(x_hbm.at[pl.ds(c * TW1 + s * SLW, SLW)], xm1_t)
        pltpu.sync_copy(xm1_t, z_s.at[sl4])
        pltpu.sync_copy(xm1_t, out_hbm.at[pl.ds(c * K * TW1 + s * SLW, SLW)])

        for k in range(1, K):
            plsc.subcore_barrier()
            pltpu.sync_copy(zzero, y_s.at[sl4])
            pltpu.sync_copy(z_s, z_t)
            plsc.subcore_barrier()

            for sup in range(0, EW1, _SUP):
                n = min(_SUP, EW1 - sup)

                def mul4(jj, carry):
                    g0 = 4 * jj
                    for u in range(4):
                        g = g0 + u
                        eidx = 4 * g + br
                        cbc = plsc.load_gather(cols_t, [eidx])
                        vbc = plsc.load_gather(vals_t, [eidx])
                        zidx = cbc * 4 + bc4
                        g16 = plsc.load_gather(z_t, [zidx])
                        gbuf[pl.ds((g - (sup // 16)) * 16, 16)] = g16 * vbc
                    return carry

                lax.fori_loop(sup // 64, (sup + n) // 64, mul4, 0)

                def fire(ci, carry):
                    pltpu.async_copy(
                        gbuf.at[pl.ds((ci - (sup // 128)) * 128, 128)],
                        y_s.at[sidx_t.at[ci]], dsem, add=True)
                    return carry

                lax.fori_loop(sup // 128, (sup + n) // 128, fire, 0)

                def drain(ci, carry):
                    pltpu.make_async_copy(
                        gbuf.at[pl.ds((ci - (sup // 128)) * 128, 128)],
                        y_s.at[sidx_t.at[ci]], dsem).wait()
                    return carry

                lax.fori_loop(sup // 128, (sup + n) // 128, drain, 0)

            plsc.subcore_barrier()
            pltpu.sync_copy(y_s.at[sl4], ybuf)

            def comb(j, carry):
                i16 = pl.ds(j * 16, 16)
                yv = ybuf[i16]
                m1 = xm1_t[i16]
                if k == 1:
                    xn = yv - m1
                else:
                    xn = 2.0 * (yv - m1) - xm2_t[i16]
                xm2_t[i16] = m1
                xm1_t[i16] = xn
                ybuf[i16] = xn
                return carry

            lax.fori_loop(0, SLW // 16, comb, 0)
            pltpu.sync_copy(ybuf, z_s.at[sl4])
            pltpu.sync_copy(
                ybuf, out_hbm.at[pl.ds((c * K + k) * TW1 + s * SLW, SLW)])

    return run(x0sc, cols2, vals2, sidx3)


def _cl1_run(x, l0_idx, l0_vals):
    x0 = x.T
    x0p = jnp.zeros((V1P, B), jnp.float32).at[:D].set(x0)
    x0sc = jnp.transpose(x0p.reshape(V1P, 2, 4), (1, 0, 2)).reshape(2 * TW1)
    ne = l0_vals.shape[0]
    ne_pad = 16 * NE1_T - ne
    rows = jnp.concatenate([l0_idx[0], jnp.zeros((ne_pad,), jnp.int32)])
    cols = jnp.concatenate([l0_idx[1], jnp.zeros((ne_pad,), jnp.int32)])
    vals = jnp.concatenate([l0_vals, jnp.zeros((ne_pad,), jnp.float32)])
    sidx = rows[:, None] * 4 + jnp.arange(4, dtype=jnp.int32)[None, :]
    pad_tgt = D * 4 + (jnp.arange(16 * NE1_T * 4, dtype=jnp.int32) % (4 * (V1P - D)))
    live = (jnp.arange(16 * NE1_T, dtype=jnp.int32) < ne)[:, None]
    sidx = jnp.where(live, sidx, pad_tgt.reshape(16 * NE1_T, 4))
    out = _cl1_sc(x0sc, cols, vals, sidx.reshape(16, NSC1, 128))
    a1 = out.reshape(2, K, V1P, 4)[:, :, :D, :]
    a1 = jnp.transpose(a1, (2, 0, 3, 1)).reshape(D * B, K)
    return a1


def _cheby_stack_xla(x0, idx, vals, k_order):
    rows = idx[0]
    cols = idx[1]

    def lop(z):
        y = jnp.zeros_like(z).at[rows].add(vals[:, None] * z[cols])
        return y - z

    xs = [x0]
    x1 = lop(x0)
    xs.append(x1)
    xm2, xm1 = x0, x1
    for _ in range(2, k_order):
        x2 = 2.0 * lop(xm1) - xm2
        xs.append(x2)
        xm2, xm1 = xm1, x2
    return jnp.stack(xs, axis=0)


def kernel(x, L0_indices, L0_values, L2_indices, L2_values, d,
           cl1_W, cl1_b, cl2_W, cl2_b, fc1_W, fc1_b, fc2_W, fc2_b):
    L0_idx = L0_indices.astype(jnp.int32)
    L2_idx = L2_indices.astype(jnp.int32)

    a1 = _cl1_run(x, L0_idx, L0_values)
    h1 = _tc_combine1(a1, cl1_W, cl1_b.reshape(1, -1))

    x0_2 = h1.reshape(V2, 32 * B)
    m = _densify_cl2(L2_idx, L2_values)
    x0p = jnp.zeros((DP, 256), jnp.float32).at[:V2].set(x0_2)
    wk = jnp.transpose(cl2_W.reshape(64, 32, K), (2, 1, 0))
    h2p = _tc_cheby_dense(m, x0p, wk, cl2_b.reshape(1, -1))

    hf = h2p.reshape(B, 625 * 64)
    return _tc_fc(hf, fc1_W, fc1_b.reshape(-1, 1), fc2_W, fc2_b.reshape(1, -1))


# speedup vs baseline: 1.2844x; 1.2844x over previous
import functools

import jax
import jax.numpy as jnp
from jax import lax
from jax.experimental import pallas as pl
from jax.experimental.pallas import tpu as pltpu
from jax.experimental.pallas import tpu_sc as plsc

D = 10000
B = 8
V2 = 2500
K = 25
LMAX = 2.0

DP = 2560
PASS_ROWS = 640
PASS_WORDS = PASS_ROWS * DP
TILE_WORDS = PASS_WORDS // 16
NE2 = V2 * 16 + V2
NE2_T = 2688
NC2 = NE2_T // 128


def _combine1_body(a_ref, w_ref, b_ref, o_ref):
    res = jax.lax.dot_general(a_ref[...], w_ref[...], (((1,), (1,)), ((), ())),
                              preferred_element_type=jnp.float32)
    res = jnp.maximum(res + b_ref[...], 0.0)
    y = res.reshape(250, 32, 32)
    m = jnp.maximum(jnp.maximum(y[:, 0:8], y[:, 8:16]),
                    jnp.maximum(y[:, 16:24], y[:, 24:32]))
    o_ref[...] = m.reshape(2000, 32)


def _tc_combine1(a, w, b):
    return pl.pallas_call(
        _combine1_body,
        grid=(10,),
        in_specs=[
            pl.BlockSpec((8000, 25), lambda i: (i, 0)),
            pl.BlockSpec((32, 25), lambda i: (0, 0)),
            pl.BlockSpec((1, 32), lambda i: (0, 0)),
        ],
        out_specs=pl.BlockSpec((2000, 32), lambda i: (i, 0)),
        out_shape=jax.ShapeDtypeStruct((20000, 32), jnp.float32),
    )(a, w, b)


def _combine2_body(a_ref, w_ref, b_ref, o_ref):
    res = jax.lax.dot_general(a_ref[...], w_ref[...], (((1,), (1,)), ((), ())),
                              preferred_element_type=jnp.float32)
    res = jnp.maximum(res + b_ref[...], 0.0)
    y = res.reshape(1000, 4, 64)
    o_ref[...] = jnp.max(y, axis=1)


def _tc_combine2(a, w, b):
    return pl.pallas_call(
        _combine2_body,
        grid=(5,),
        in_specs=[
            pl.BlockSpec((4000, 800), lambda i: (i, 0)),
            pl.BlockSpec((64, 800), lambda i: (0, 0)),
            pl.BlockSpec((1, 64), lambda i: (0, 0)),
        ],
        out_specs=pl.BlockSpec((1000, 64), lambda i: (i, 0)),
        out_shape=jax.ShapeDtypeStruct((5000, 64), jnp.float32),
    )(a, w, b)


def _fc_body(h_ref, w1_ref, b1_ref, w2_ref, b2_ref, o_ref, zt_ref):
    i = pl.program_id(0)
    zt = jnp.maximum(jax.lax.dot_general(
        w1_ref[...], h_ref[...], (((1,), (1,)), ((), ())),
        preferred_element_type=jnp.float32) + b1_ref[...], 0.0)
    zt_ref[pl.ds(i * 64, 64), :] = zt

    @pl.when(i == pl.num_programs(0) - 1)
    def _():
        o_ref[...] = jax.lax.dot_general(
            zt_ref[...], w2_ref[...], (((0,), (1,)), ((), ())),
            preferred_element_type=jnp.float32) + b2_ref[...]


def _tc_fc(h, w1, b1, w2, b2):
    from jax.experimental.pallas import tpu as pltpu
    return pl.pallas_call(
        _fc_body,
        grid=(8,),
        in_specs=[
            pl.BlockSpec((8, 40000), lambda i: (0, 0)),
            pl.BlockSpec((64, 40000), lambda i: (i, 0)),
            pl.BlockSpec((64, 1), lambda i: (i, 0)),
            pl.BlockSpec((10, 512), lambda i: (0, 0)),
            pl.BlockSpec((1, 10), lambda i: (0, 0)),
        ],
        out_specs=pl.BlockSpec((8, 10), lambda i: (0, 0)),
        out_shape=jax.ShapeDtypeStruct((8, 10), jnp.float32),
        scratch_shapes=[pltpu.VMEM((512, 8), jnp.float32)],
    )(h, w1, b1, w2, b2)


def _densify_sc(li_all, vals_tiled, zeros_flat):
    mesh = plsc.VectorSubcoreMesh(core_axis_name="c", subcore_axis_name="s")

    @functools.partial(
        pl.kernel, mesh=mesh,
        out_type=jax.ShapeDtypeStruct((DP * DP,), jnp.float32),
        scratch_types=[
            pltpu.VMEM((NC2, 128), jnp.int32),
            pltpu.VMEM((NE2_T,), jnp.float32),
            pltpu.VMEM_SHARED((PASS_WORDS,), jnp.float32),
        ],
    )
    def run(li_hbm, vals_hbm, zeros_hbm, m_hbm, libuf, vbuf, spbuf):
        c = lax.axis_index("c")
        s = lax.axis_index("s")
        pltpu.sync_copy(vals_hbm.at[s], vbuf)
        tile_off = s * TILE_WORDS
        for p_local in range(2):
            pglob = 2 * c + p_local
            pltpu.sync_copy(zeros_hbm.at[pl.ds(tile_off, TILE_WORDS)],
                            spbuf.at[pl.ds(tile_off, TILE_WORDS)])
            pltpu.sync_copy(li_hbm.at[pglob, s], libuf)
            plsc.subcore_barrier()

            def scat(ci, carry):
                pltpu.sync_copy(vbuf.at[pl.ds(ci * 128, 128)],
                                spbuf.at[libuf.at[ci]], add=True)
                return carry

            lax.fori_loop(0, NC2, scat, 0)
            plsc.subcore_barrier()
            pltpu.sync_copy(
                spbuf.at[pl.ds(tile_off, TILE_WORDS)],
                m_hbm.at[pl.ds(pglob * PASS_WORDS + tile_off, TILE_WORDS)])
            plsc.subcore_barrier()

    return run(li_all, vals_tiled, zeros_flat)


def _densify_cl2(l2_idx, l2_vals):
    rows = jnp.concatenate([l2_idx[0], jnp.arange(V2, dtype=jnp.int32)])
    cols = jnp.concatenate([l2_idx[1], jnp.arange(V2, dtype=jnp.int32)])
    vals = jnp.concatenate([l2_vals, jnp.full((V2,), -1.0, jnp.float32)])
    npad = 16 * NE2_T - NE2
    rows = jnp.concatenate([rows, jnp.zeros((npad,), jnp.int32)])
    cols = jnp.concatenate([cols, jnp.zeros((npad,), jnp.int32)])
    vals = jnp.concatenate([vals, jnp.zeros((npad,), jnp.float32)])
    tile_id = (jnp.arange(16 * NE2_T, dtype=jnp.int32) // NE2_T) % 16
    base = (jnp.arange(4, dtype=jnp.int32) * PASS_ROWS)[:, None]
    in_pass = (rows[None, :] >= base) & (rows[None, :] < base + PASS_ROWS)
    li = jnp.where(in_pass, (rows[None, :] - base) * DP + cols[None, :],
                   2512 + tile_id[None, :])
    li_all = li.reshape(4, 16, NC2, 128)
    vals_tiled = vals.reshape(16, NE2_T)
    zeros_flat = jnp.zeros((PASS_WORDS,), jnp.float32)
    m_flat = _densify_sc(li_all, vals_tiled, zeros_flat)
    return m_flat.reshape(DP, DP)


_CH = 512


def _cheby_dense_body(lh_ref, ll_ref, x0_ref, wk_ref, wklast_ref, b2_ref,
                      o_ref, xa_ref, xb_ref, xh_ref, xl_ref, acc_ref):
    p = pl.program_id(0)
    dims = (((1,), (0,)), ((), ()))

    def split(src_ref):
        for r in range(DP // _CH):
            sl = pl.ds(r * _CH, _CH)
            chunk = src_ref[sl, :]
            h = chunk.astype(jnp.bfloat16)
            xh_ref[sl, :] = h
            xl_ref[sl, :] = (chunk - h.astype(jnp.float32)).astype(jnp.bfloat16)

    def mm_rows(r):
        sl = pl.ds(r * _CH, _CH)
        acc = jax.lax.dot_general(lh_ref[sl, :], xh_ref[...], dims,
                                  preferred_element_type=jnp.float32)
        acc += jax.lax.dot_general(lh_ref[sl, :], xl_ref[...], dims,
                                   preferred_element_type=jnp.float32)
        acc += jax.lax.dot_general(ll_ref[sl, :], xh_ref[...], dims,
                                   preferred_element_type=jnp.float32)
        return acc

    def add_contrib(x, w3):
        acc_ref[...] += jax.lax.dot_general(
            x, w3[0], dims, preferred_element_type=jnp.float32)

    @pl.when(p == 0)
    def _():
        acc_ref[...] = jnp.zeros_like(acc_ref)
        x0 = x0_ref[...]
        xa_ref[...] = x0
        add_contrib(x0, wk_ref[...])
        split(x0_ref)
        for r in range(DP // _CH):
            sl = pl.ds(r * _CH, _CH)
            xn = mm_rows(r)
            xb_ref[sl, :] = xn

    @pl.when(p % 2 == 1)
    def _():
        add_contrib(xb_ref[...], wk_ref[...])
        split(xb_ref)
        for r in range(DP // _CH):
            sl = pl.ds(r * _CH, _CH)
            xn = 2.0 * mm_rows(r) - xa_ref[sl, :]
            xa_ref[sl, :] = xn

    @pl.when((p % 2 == 0) & (p > 0))
    def _():
        add_contrib(xa_ref[...], wk_ref[...])
        split(xa_ref)
        for r in range(DP // _CH):
            sl = pl.ds(r * _CH, _CH)
            xn = 2.0 * mm_rows(r) - xb_ref[sl, :]
            xb_ref[sl, :] = xn

    @pl.when(p == K - 2)
    def _():
        add_contrib(xa_ref[...], wklast_ref[...])
        for b in range(B):
            z = jnp.maximum(
                acc_ref[:V2, b * 64:(b + 1) * 64] + b2_ref[...], 0.0)
            o_ref[b] = jnp.max(z.reshape(V2 // 4, 4, 64), axis=1)


def _tc_cheby_dense(m, x0, wk, b2):
    mh = m.astype(jnp.bfloat16)
    ml = (m - mh.astype(jnp.float32)).astype(jnp.bfloat16)
    eye8 = jnp.eye(B, dtype=jnp.float32)
    wblk = (eye8[None, :, None, :, None] * wk[:, None, :, None, :]
            ).reshape(K, 256, 512)
    return pl.pallas_call(
        _cheby_dense_body,
        grid=(K - 1,),
        in_specs=[
            pl.BlockSpec((DP, DP), lambda p: (0, 0)),
            pl.BlockSpec((DP, DP), lambda p: (0, 0)),
            pl.BlockSpec((DP, 256), lambda p: (0, 0)),
            pl.BlockSpec((1, 256, 512), lambda p: (p, 0, 0)),
            pl.BlockSpec((1, 256, 512), lambda p: (K - 1, 0, 0)),
            pl.BlockSpec((1, 64), lambda p: (0, 0)),
        ],
        out_specs=pl.BlockSpec((B, V2 // 4, 64), lambda p: (0, 0, 0)),
        out_shape=jax.ShapeDtypeStruct((B, V2 // 4, 64), jnp.float32),
        scratch_shapes=[pltpu.VMEM((DP, 256), jnp.float32),
                        pltpu.VMEM((DP, 256), jnp.float32),
                        pltpu.VMEM((DP, 256), jnp.bfloat16),
                        pltpu.VMEM((DP, 256), jnp.bfloat16),
                        pltpu.VMEM((DP, 512), jnp.float32)],
    )(mh, ml, x0, wblk, wblk, b2)


V1P = 10112
RT1 = V1P // 16
SLW = RT1 * 4
TW1 = V1P * 4
NE1_T = 10112
EW1 = NE1_T * 4
NSC1 = EW1 // 128
_SUP = 8192


def _cl1_sc(x0sc, cols2, vals2, sidx3):
    mesh = plsc.VectorSubcoreMesh(core_axis_name="c", subcore_axis_name="s")

    @functools.partial(
        pl.kernel, mesh=mesh,
        out_type=jax.ShapeDtypeStruct((2 * K * TW1,), jnp.float32),
        compiler_params=pltpu.CompilerParams(needs_layout_passes=False),
        scratch_types=[
            pltpu.VMEM((NE1_T,), jnp.int32),
            pltpu.VMEM((NE1_T,), jnp.float32),
            pltpu.VMEM((NSC1, 128), jnp.int32),
            pltpu.VMEM((TW1,), jnp.float32),
            pltpu.VMEM((_SUP,), jnp.float32),
            pltpu.VMEM((SLW,), jnp.float32),
            pltpu.VMEM((SLW,), jnp.float32),
            pltpu.VMEM((SLW,), jnp.float32),
            pltpu.VMEM((SLW,), jnp.float32),
            pltpu.VMEM_SHARED((TW1,), jnp.float32),
            pltpu.VMEM_SHARED((TW1,), jnp.float32),
            pltpu.SemaphoreType.DMA,
        ],
    )
    def run(x_hbm, cols_hbm, vals_hbm, sidx_hbm, out_hbm,
            cols_t, vals_t, sidx_t, z_t, gbuf, ybuf, xm1_t, xm2_t, zzero,
            z_s, y_s, dsem):
        c = lax.axis_index("c")
        s = lax.axis_index("s")
        sl4 = pl.ds(s * SLW, SLW)

        iota16 = lax.iota(jnp.int32, 16)
        br = iota16 // 4
        bc4 = iota16 % 4
        zero16 = jnp.zeros((16,), jnp.float32)

        pltpu.sync_copy(cols_hbm.at[pl.ds(s * NE1_T, NE1_T)], cols_t)
        pltpu.sync_copy(vals_hbm.at[pl.ds(s * NE1_T, NE1_T)], vals_t)
        pltpu.sync_copy(sidx_hbm.at[s], sidx_t)

        def zinit(j, carry):
            zzero[pl.ds(j * 16, 16)] = zero16
            return carry
        lax.fori_loop(0, SLW // 16, zinit, 0)

        pltpu.sync_copy(x_hbm.at[pl.ds(c * TW1 + s * SLW, SLW)], xm1_t)
        pltpu.sync_copy(xm1_t, z_s.at[sl4])
        pltpu.sync_copy(xm1_t, out_hbm.at[pl.ds(c * K * TW1 + s * SLW, SLW)])

        for k in range(1, K):
            plsc.subcore_barrier()
            pltpu.sync_copy(zzero, y_s.at[sl4])
            pltpu.sync_copy(z_s, z_t)
            plsc.subcore_barrier()

            for sup in range(0, EW1, _SUP):
                n = min(_SUP, EW1 - sup)

                def mul4(jj, carry):
                    g0 = 4 * jj
                    for u in range(4):
                        g = g0 + u
                        eidx = 4 * g + br
                        cbc = plsc.load_gather(cols_t, [eidx])
                        vbc = plsc.load_gather(vals_t, [eidx])
                        zidx = cbc * 4 + bc4
                        g16 = plsc.load_gather(z_t, [zidx])
                        gbuf[pl.ds((g - (sup // 16)) * 16, 16)] = g16 * vbc
                    return carry

                lax.fori_loop(sup // 64, (sup + n) // 64, mul4, 0)

                def fire(ci, carry):
                    pltpu.async_copy(
                        gbuf.at[pl.ds((ci - (sup // 128)) * 128, 128)],
                        y_s.at[sidx_t.at[ci]], dsem, add=True)
                    return carry

                lax.fori_loop(sup // 128, (sup + n) // 128, fire, 0)

                def drain(ci, carry):
                    pltpu.make_async_copy(
                        gbuf.at[pl.ds((ci - (sup // 128)) * 128, 128)],
                        y_s.at[sidx_t.at[ci]], dsem).wait()
                    return carry

                lax.fori_loop(sup // 128, (sup + n) // 128, drain, 0)

            plsc.subcore_barrier()
            pltpu.sync_copy(y_s.at[sl4], ybuf)

            def comb(j, carry):
                i16 = pl.ds(j * 16, 16)
                yv = ybuf[i16]
                m1 = xm1_t[i16]
                if k == 1:
                    xn = yv - m1
                else:
                    xn = 2.0 * (yv - m1) - xm2_t[i16]
                xm2_t[i16] = m1
                xm1_t[i16] = xn
                ybuf[i16] = xn
                return carry

            lax.fori_loop(0, SLW // 16, comb, 0)
            pltpu.sync_copy(ybuf, z_s.at[sl4])
            pltpu.sync_copy(
                ybuf, out_hbm.at[pl.ds((c * K + k) * TW1 + s * SLW, SLW)])

    return run(x0sc, cols2, vals2, sidx3)


def _cl1_run(x, l0_idx, l0_vals):
    x0 = x.T
    x0p = jnp.zeros((V1P, B), jnp.float32).at[:D].set(x0)
    x0sc = jnp.transpose(x0p.reshape(V1P, 2, 4), (1, 0, 2)).reshape(2 * TW1)
    ne = l0_vals.shape[0]
    ne_pad = 16 * NE1_T - ne
    rows = jnp.concatenate([l0_idx[0], jnp.zeros((ne_pad,), jnp.int32)])
    cols = jnp.concatenate([l0_idx[1], jnp.zeros((ne_pad,), jnp.int32)])
    vals = jnp.concatenate([l0_vals, jnp.zeros((ne_pad,), jnp.float32)])
    sidx = rows[:, None] * 4 + jnp.arange(4, dtype=jnp.int32)[None, :]
    pad_tgt = D * 4 + (jnp.arange(16 * NE1_T * 4, dtype=jnp.int32) % (4 * (V1P - D)))
    live = (jnp.arange(16 * NE1_T, dtype=jnp.int32) < ne)[:, None]
    sidx = jnp.where(live, sidx, pad_tgt.reshape(16 * NE1_T, 4))
    out = _cl1_sc(x0sc, cols, vals, sidx.reshape(16, NSC1, 128))
    a1 = out.reshape(2, K, V1P, 4)[:, :, :D, :]
    a1 = jnp.transpose(a1, (2, 0, 3, 1)).reshape(D * B, K)
    return a1


def _cheby_stack_xla(x0, idx, vals, k_order):
    rows = idx[0]
    cols = idx[1]

    def lop(z):
        y = jnp.zeros_like(z).at[rows].add(vals[:, None] * z[cols])
        return y - z

    xs = [x0]
    x1 = lop(x0)
    xs.append(x1)
    xm2, xm1 = x0, x1
    for _ in range(2, k_order):
        x2 = 2.0 * lop(xm1) - xm2
        xs.append(x2)
        xm2, xm1 = xm1, x2
    return jnp.stack(xs, axis=0)


def kernel(x, L0_indices, L0_values, L2_indices, L2_values, d,
           cl1_W, cl1_b, cl2_W, cl2_b, fc1_W, fc1_b, fc2_W, fc2_b):
    L0_idx = L0_indices.astype(jnp.int32)
    L2_idx = L2_indices.astype(jnp.int32)

    a1 = _cl1_run(x, L0_idx, L0_values)
    h1 = _tc_combine1(a1, cl1_W, cl1_b.reshape(1, -1))

    x0_2 = h1.reshape(V2, 32 * B)
    m = _densify_cl2(L2_idx, L2_values)
    x0p = jnp.zeros((DP, 256), jnp.float32).at[:V2].set(x0_2)
    wk = jnp.transpose(cl2_W.reshape(64, 32, K), (2, 1, 0))
    h2p = _tc_cheby_dense(m, x0p, wk, cl2_b.reshape(1, -1))

    hf = h2p.reshape(B, 625 * 64)
    return _tc_fc(hf, fc1_W, fc1_b.reshape(-1, 1), fc2_W, fc2_b.reshape(1, -1))
